# Initial kernel scaffold; baseline (speedup 1.0000x reference)
import jax
import jax.numpy as jnp
from jax.experimental import pallas as pl


def kernel(unique_params, index_map):
    raise NotImplementedError("write your pallas kernel here")


# speedup vs baseline: 1.5351x; 1.5351x over previous
import functools

import jax
import jax.numpy as jnp
from jax import lax
from jax.experimental import pallas as pl
from jax.experimental.pallas import tpu as pltpu
from jax.experimental.pallas import tpu_sc as plsc

_NC = 2
_NS = 16
_NW = _NC * _NS


def _ceil_to(x, m):
    return (x + m - 1) // m * m


@functools.partial(jax.jit, static_argnames=("n_rows", "dim", "chunk"))
def _gather_rows(table2d, idx_pad, n_rows, dim, chunk):
    full_chunks = n_rows // chunk
    tail = n_rows - full_chunks * chunk
    tail_pad = _ceil_to(tail, 8)
    rounds = (full_chunks + _NW - 1) // _NW

    mesh = plsc.VectorSubcoreMesh(
        core_axis_name="c", subcore_axis_name="s",
        num_cores=_NC, num_subcores=_NS)

    scratch = [
        pltpu.VMEM((chunk,), jnp.int32),
        pltpu.VMEM((chunk, dim), table2d.dtype),
        pltpu.SemaphoreType.DMA,
    ]
    if tail:
        scratch += [
            pltpu.VMEM((tail_pad,), jnp.int32),
            pltpu.VMEM((tail_pad, dim), table2d.dtype),
        ]

    @functools.partial(
        pl.kernel,
        out_type=jax.ShapeDtypeStruct((n_rows, dim), table2d.dtype),
        mesh=mesh,
        scratch_types=scratch,
    )
    def run(table_hbm, idx_hbm, out_hbm, idx_v, rows_v, sem, *tail_scratch):
        wid = lax.axis_index("s") * _NC + lax.axis_index("c")

        def one_round(r, carry):
            g = r * _NW + wid

            @pl.when(g < full_chunks)
            def _():
                base = g * chunk
                pltpu.sync_copy(idx_hbm.at[pl.ds(base, chunk)], idx_v)
                pltpu.async_copy(table_hbm.at[idx_v], rows_v, sem).wait()
                pltpu.sync_copy(rows_v, out_hbm.at[pl.ds(base, chunk)])

            return carry

        lax.fori_loop(0, rounds, one_round, 0)

        if tail:
            idx_t, rows_t = tail_scratch
            tbase = full_chunks * chunk

            @pl.when(wid == _NW - 1)
            def _():
                pltpu.sync_copy(idx_hbm.at[pl.ds(tbase, tail_pad)], idx_t)
                pltpu.async_copy(table_hbm.at[idx_t], rows_t, sem).wait()
                pltpu.sync_copy(rows_t.at[pl.ds(0, tail)],
                                out_hbm.at[pl.ds(tbase, tail)])

    return run(table2d, idx_pad)


def kernel(unique_params, index_map):
    n, in_dim, out_dim = unique_params.shape
    dim = in_dim * out_dim
    b = index_map.size
    table2d = unique_params.reshape(n, dim)
    idx = index_map.reshape(-1).astype(jnp.int32)
    b_pad = _ceil_to(b, 8)
    if b_pad != b:
        idx = jnp.pad(idx, (0, b_pad - b))
    out2d = _gather_rows(table2d, idx, n_rows=b, dim=dim, chunk=256)
    return out2d.reshape(*index_map.shape, in_dim, out_dim)
